# baseline (speedup 1.0000x reference)
import jax
import jax.numpy as jnp
from jax import lax
from jax.experimental import pallas as pl
from jax.experimental.pallas import tpu as pltpu
from jax.experimental.pallas import tpu_sc as plsc

B = 16384
L = 200
NC = 2
NS = 16
NW = NC * NS
ROWS_PER_TILE = B // NW
WPR = L // 4
TILE_WORDS = ROWS_PER_TILE * WPR
NCHUNK = 4
CHUNK_WORDS = TILE_WORDS // NCHUNK
GROUPS_PER_CHUNK = ROWS_PER_TILE // 16 // NCHUNK


def _body(inputs_hbm, table_hbm, w_hbm, b_hbm, out_hbm,
          idx_v, out_v, tab_v, w_v, b_v, lut_v, sem0, sem1, sem2, sem3):
    wid = lax.axis_index("s") * NC + lax.axis_index("c")
    basew = wid * TILE_WORDS

    sems = [sem0, sem1, sem2, sem3]
    cps = []
    for k in range(NCHUNK):
        cps.append(pltpu.async_copy(
            inputs_hbm.at[pl.ds(basew + k * CHUNK_WORDS, CHUNK_WORDS)],
            idx_v.at[pl.ds(k * CHUNK_WORDS, CHUNK_WORDS)], sems[k]))
    pltpu.sync_copy(table_hbm, tab_v)
    pltpu.sync_copy(w_hbm, w_v)
    pltpu.sync_copy(b_hbm, b_v)

    lane = lax.iota(jnp.int32, 16)
    vrow = jnp.minimum(lane, 9)
    t0 = plsc.load_gather(tab_v, [vrow * 3])
    t1 = plsc.load_gather(tab_v, [vrow * 3 + 1])
    t2 = plsc.load_gather(tab_v, [vrow * 3 + 2])
    wv = w_v[...]
    lut = (t0 * wv[0] + t1 * wv[1] + t2 * wv[2]) * jnp.float32(1.0 / L)
    for v in range(10):
        lut_v[pl.ds(v * 16, 16)] = jnp.full((16,), lut[v], jnp.float32)

    bias = jnp.full((16,), b_v[...][0], jnp.float32)
    offs = [(lane + d) & 15 for d in range(16)]

    def addword(word, acc_w):
        a0 = (word << 4) & 0xFF0
        a1 = (word >> 4) & 0xFF0
        a2 = (word >> 12) & 0xFF0
        a3 = (word >> 20) & 0xFF0
        acc_w = acc_w + plsc.load_gather(lut_v, [a0 + lane])
        acc_w = acc_w + plsc.load_gather(lut_v, [a1 + lane])
        acc_w = acc_w + plsc.load_gather(lut_v, [a2 + lane])
        acc_w = acc_w + plsc.load_gather(lut_v, [a3 + lane])
        return acc_w

    def group(g, carry):
        rowbase = (g * 16 + lane) * WPR
        acc = bias

        def window(w, acc_w):
            rb_w = rowbase + w * 16
            for d in range(16):
                word = plsc.load_gather(idx_v, [rb_w + offs[d]])
                acc_w = addword(word, acc_w)
            return acc_w

        acc = lax.fori_loop(0, WPR // 16, window, acc)
        rb_t = rowbase + (WPR // 16) * 16
        for d in range(WPR % 16):
            word = plsc.load_gather(idx_v, [rb_t + (offs[d] & (WPR % 16 - 1))])
            acc = addword(word, acc)
        out_v[pl.ds(g * 16, 16)] = acc
        return carry

    for k in range(NCHUNK):
        cps[k].wait()
        lax.fori_loop(k * GROUPS_PER_CHUNK, (k + 1) * GROUPS_PER_CHUNK,
                      group, 0)

    pltpu.sync_copy(out_v, out_hbm.at[pl.ds(wid * ROWS_PER_TILE,
                                            ROWS_PER_TILE)])


@jax.jit
def _run(packed_words, table_flat, w_pad, b_pad):
    mesh = plsc.VectorSubcoreMesh(core_axis_name="c", subcore_axis_name="s")
    fn = pl.kernel(
        _body,
        out_type=jax.ShapeDtypeStruct((B,), jnp.float32),
        mesh=mesh,
        scratch_types=[
            pltpu.VMEM((TILE_WORDS,), jnp.int32),
            pltpu.VMEM((ROWS_PER_TILE,), jnp.float32),
            pltpu.VMEM((30,), jnp.float32),
            pltpu.VMEM((16,), jnp.float32),
            pltpu.VMEM((16,), jnp.float32),
            pltpu.VMEM((160,), jnp.float32),
            pltpu.SemaphoreType.DMA,
            pltpu.SemaphoreType.DMA,
            pltpu.SemaphoreType.DMA,
            pltpu.SemaphoreType.DMA,
        ],
        compiler_params=pltpu.CompilerParams(needs_layout_passes=False),
    )
    return fn(packed_words, table_flat, w_pad, b_pad)


def kernel(inputs, table, W, b):
    packed = lax.bitcast_convert_type(
        inputs.astype(jnp.int8).reshape(-1, 4), jnp.int32)
    w_pad = jnp.zeros((16,), jnp.float32).at[:3].set(W.reshape(-1))
    b_pad = jnp.zeros((16,), jnp.float32).at[:1].set(b)
    out = _run(packed, table.reshape(-1), w_pad, b_pad)
    return out.reshape(B, 1)


# speedup vs baseline: 3.0816x; 3.0816x over previous
import jax
import jax.numpy as jnp
from jax import lax
from jax.experimental import pallas as pl
from jax.experimental.pallas import tpu as pltpu
from jax.experimental.pallas import tpu_sc as plsc

B = 16384
L = 200
NC = 2
NS = 16
NW = NC * NS
ROWS_PER_TILE = B // NW
WPR = L // 4
TILE_WORDS = ROWS_PER_TILE * WPR
NCHUNK = 4
CHUNK_WORDS = TILE_WORDS // NCHUNK
GROUPS_PER_CHUNK = ROWS_PER_TILE // 16 // NCHUNK


def _body(inputs_hbm, table_hbm, w_hbm, b_hbm, out_hbm,
          idx_v, out_v, tab_v, w_v, b_v, lut_v, sem0, sem1, sem2, sem3):
    wid = lax.axis_index("s") * NC + lax.axis_index("c")
    basew = wid * TILE_WORDS

    sems = [sem0, sem1, sem2, sem3]
    cps = []
    for k in range(NCHUNK):
        cps.append(pltpu.async_copy(
            inputs_hbm.at[pl.ds(basew + k * CHUNK_WORDS, CHUNK_WORDS)],
            idx_v.at[pl.ds(k * CHUNK_WORDS, CHUNK_WORDS)], sems[k]))
    pltpu.sync_copy(table_hbm, tab_v)
    pltpu.sync_copy(w_hbm, w_v)
    pltpu.sync_copy(b_hbm, b_v)

    lane = lax.iota(jnp.int32, 16)
    vrow = jnp.minimum(lane, 9)
    t0 = plsc.load_gather(tab_v, [vrow * 3])
    t1 = plsc.load_gather(tab_v, [vrow * 3 + 1])
    t2 = plsc.load_gather(tab_v, [vrow * 3 + 2])
    wv = w_v[...]
    lut = (t0 * wv[0] + t1 * wv[1] + t2 * wv[2]) * jnp.float32(1.0 / L)
    for v in range(10):
        lut_v[pl.ds(v * 16, 16)] = jnp.full((16,), lut[v], jnp.float32)

    bias = jnp.full((16,), b_v[...][0], jnp.float32)
    offs = [(lane + d) & 15 for d in range(16)]

    def addword(word, acc_w):
        a0 = (word << 4) & 0xFF0
        a1 = (word >> 4) & 0xFF0
        a2 = (word >> 12) & 0xFF0
        a3 = (word >> 20) & 0xFF0
        acc_w = acc_w + plsc.load_gather(lut_v, [a0 + lane])
        acc_w = acc_w + plsc.load_gather(lut_v, [a1 + lane])
        acc_w = acc_w + plsc.load_gather(lut_v, [a2 + lane])
        acc_w = acc_w + plsc.load_gather(lut_v, [a3 + lane])
        return acc_w

    def group(g, carry):
        rowbase = (g * 16 + lane) * WPR
        acc = bias

        def window(w, acc_w):
            rb_w = rowbase + w * 16
            for d in range(16):
                word = plsc.load_gather(idx_v, [rb_w + offs[d]])
                acc_w = addword(word, acc_w)
            return acc_w

        acc = lax.fori_loop(0, WPR // 16, window, acc)
        rb_t = rowbase + (WPR // 16) * 16
        for d in range(WPR % 16):
            word = plsc.load_gather(idx_v, [rb_t + (offs[d] & (WPR % 16 - 1))])
            acc = addword(word, acc)
        out_v[pl.ds(g * 16, 16)] = acc
        return carry

    for k in range(NCHUNK):
        cps[k].wait()
        lax.fori_loop(k * GROUPS_PER_CHUNK, (k + 1) * GROUPS_PER_CHUNK,
                      group, 0)

    pltpu.sync_copy(out_v, out_hbm.at[pl.ds(wid * ROWS_PER_TILE,
                                            ROWS_PER_TILE)])


@jax.jit
def _run(packed_words, table_flat, w_pad, b_pad):
    mesh = plsc.VectorSubcoreMesh(core_axis_name="c", subcore_axis_name="s")
    fn = pl.kernel(
        _body,
        out_type=jax.ShapeDtypeStruct((B,), jnp.float32),
        mesh=mesh,
        scratch_types=[
            pltpu.VMEM((TILE_WORDS,), jnp.int32),
            pltpu.VMEM((ROWS_PER_TILE,), jnp.float32),
            pltpu.VMEM((30,), jnp.float32),
            pltpu.VMEM((16,), jnp.float32),
            pltpu.VMEM((16,), jnp.float32),
            pltpu.VMEM((160,), jnp.float32),
            pltpu.SemaphoreType.DMA,
            pltpu.SemaphoreType.DMA,
            pltpu.SemaphoreType.DMA,
            pltpu.SemaphoreType.DMA,
        ],
        compiler_params=pltpu.CompilerParams(needs_layout_passes=False),
    )
    return fn(packed_words, table_flat, w_pad, b_pad)


def kernel(inputs, table, W, b):
    r = inputs.astype(jnp.int32).reshape(B, WPR, 4)
    packed = (r[:, :, 0] | (r[:, :, 1] << 8) | (r[:, :, 2] << 16)
              | (r[:, :, 3] << 24)).reshape(-1)
    w_pad = jnp.zeros((16,), jnp.float32).at[:3].set(W.reshape(-1))
    b_pad = jnp.zeros((16,), jnp.float32).at[:1].set(b)
    out = _run(packed, table.reshape(-1), w_pad, b_pad)
    return out.reshape(B, 1)


# speedup vs baseline: 5.8627x; 1.9025x over previous
import jax
import jax.numpy as jnp
from jax import lax
from jax.experimental import pallas as pl
from jax.experimental.pallas import tpu as pltpu
from jax.experimental.pallas import tpu_sc as plsc

B = 16384
L = 200
NC = 2
NS = 16
NW = NC * NS
ROWS_PER_TILE = B // NW
TILE_WORDS = ROWS_PER_TILE * L
DROW = 128
DMA_ROWS = TILE_WORDS // DROW
NCHUNK = 4
CHUNK_ROWS = DMA_ROWS // NCHUNK
SPLITS = (104, 96)
GROUPS = ROWS_PER_TILE // 16
GROUPS_PER_CHUNK = GROUPS // NCHUNK


def _body(inputs_hbm, table_hbm, w_hbm, b_hbm, out_hbm,
          slab_v, out_v, tab_v, w_v, b_v, lut_v, ilist_v,
          sem0, sem1, sem2, sem3):
    wid = lax.axis_index("s") * NC + lax.axis_index("c")
    rowbase_dma = wid * DMA_ROWS

    lane = lax.iota(jnp.int32, 16)
    for j in range(DMA_ROWS // 16):
        ilist_v[pl.ds(j * 16, 16)] = rowbase_dma + j * 16 + lane

    sems = [sem0, sem1, sem2, sem3]
    cps = []
    for k in range(NCHUNK):
        sub = []
        off = k * CHUNK_ROWS
        for n in SPLITS:
            sub.append(pltpu.async_copy(
                inputs_hbm.at[ilist_v.at[pl.ds(off, n)]],
                slab_v.at[pl.ds(off, n)], sems[k]))
            off += n
        cps.append(sub)
    pltpu.sync_copy(table_hbm, tab_v)
    pltpu.sync_copy(w_hbm, w_v)
    pltpu.sync_copy(b_hbm, b_v)

    vrow = jnp.minimum(lane, 9)
    t0 = plsc.load_gather(tab_v, [vrow * 3])
    t1 = plsc.load_gather(tab_v, [vrow * 3 + 1])
    t2 = plsc.load_gather(tab_v, [vrow * 3 + 2])
    wv = w_v[...]
    lut = (t0 * wv[0] + t1 * wv[1] + t2 * wv[2]) * jnp.float32(1.0 / L)
    for v in range(10):
        lut_v[pl.ds(v * 16, 16)] = jnp.full((16,), lut[v], jnp.float32)

    bias = jnp.full((16,), b_v[...][0], jnp.float32)
    offs = [(lane + d) & 15 for d in range(16)]

    def group(g, carry):
        rowbase = (g * 16 + lane) * L
        acc = bias

        def window(w, acc_w):
            rb_w = rowbase + w * 16
            for d in range(16):
                a = rb_w + offs[d]
                vi = plsc.load_gather(slab_v, [a >> 7, a & 127])
                acc_w = acc_w + plsc.load_gather(lut_v, [vi * 16 + lane])
            return acc_w

        acc = lax.fori_loop(0, L // 16, window, acc)
        rb_t = rowbase + (L // 16) * 16
        for d in range(L % 16):
            a = rb_t + (offs[d] & 7)
            vi = plsc.load_gather(slab_v, [a >> 7, a & 127])
            acc = acc + plsc.load_gather(lut_v, [vi * 16 + lane])
        out_v[pl.ds(g * 16, 16)] = acc
        return carry

    for k in range(NCHUNK):
        for cp in cps[k]:
            cp.wait()
        lax.fori_loop(k * GROUPS_PER_CHUNK, (k + 1) * GROUPS_PER_CHUNK,
                      group, 0)

    pltpu.sync_copy(out_v, out_hbm.at[pl.ds(wid * ROWS_PER_TILE,
                                            ROWS_PER_TILE)])


@jax.jit
def _run(inputs2d, table_flat, w_pad, b_pad):
    mesh = plsc.VectorSubcoreMesh(core_axis_name="c", subcore_axis_name="s")
    fn = pl.kernel(
        _body,
        out_type=jax.ShapeDtypeStruct((B,), jnp.float32),
        mesh=mesh,
        scratch_types=[
            pltpu.VMEM((DMA_ROWS, DROW), jnp.int32),
            pltpu.VMEM((ROWS_PER_TILE,), jnp.float32),
            pltpu.VMEM((30,), jnp.float32),
            pltpu.VMEM((16,), jnp.float32),
            pltpu.VMEM((16,), jnp.float32),
            pltpu.VMEM((160,), jnp.float32),
            pltpu.VMEM((DMA_ROWS,), jnp.int32),
            pltpu.SemaphoreType.DMA,
            pltpu.SemaphoreType.DMA,
            pltpu.SemaphoreType.DMA,
            pltpu.SemaphoreType.DMA,
        ],
        compiler_params=pltpu.CompilerParams(needs_layout_passes=False),
    )
    return fn(inputs2d, table_flat, w_pad, b_pad)


def kernel(inputs, table, W, b):
    inputs2d = inputs.astype(jnp.int32).reshape(B * L // DROW, DROW)
    w_pad = jnp.zeros((16,), jnp.float32).at[:3].set(W.reshape(-1))
    b_pad = jnp.zeros((16,), jnp.float32).at[:1].set(b)
    out = _run(inputs2d, table.reshape(-1), w_pad, b_pad)
    return out.reshape(B, 1)


# speedup vs baseline: 7.9941x; 1.3636x over previous
import numpy as np

import jax
import jax.numpy as jnp
from jax import lax
from jax.experimental import pallas as pl
from jax.experimental.pallas import tpu as pltpu
from jax.experimental.pallas import tpu_sc as plsc

B = 16384
L = 200
NC = 2
NS = 16
NW = NC * NS
ROWS_PER_TILE = B // NW
WPR = L // 4
TILE_WORDS = ROWS_PER_TILE * WPR
DROW = 128
DMA_ROWS = TILE_WORDS // DROW
SPLITS = (104, 96)
ILIST_PAD = 208
GROUPS = ROWS_PER_TILE // 16

_PACK = np.kron(np.eye(WPR, dtype=np.float32),
                np.array([1.0, 16.0, 256.0, 4096.0], np.float32)).T


def _body(inputs_hbm, table_hbm, w_hbm, b_hbm, out_hbm,
          slab_v, out_v, tab_v, w_v, b_v, lut_v, ilist_v, sem):
    wid = lax.axis_index("s") * NC + lax.axis_index("c")
    rowbase_dma = wid * DMA_ROWS

    lane = lax.iota(jnp.int32, 16)
    for j in range(ILIST_PAD // 16):
        ilist_v[pl.ds(j * 16, 16)] = jnp.minimum(
            rowbase_dma + j * 16 + lane, B * WPR // DROW - 1)

    cps = []
    off = 0
    for n in SPLITS:
        cps.append(pltpu.async_copy(
            inputs_hbm.at[ilist_v.at[pl.ds(off, n)]],
            slab_v.at[pl.ds(off, n)], sem))
        off += n
    pltpu.sync_copy(table_hbm, tab_v)
    pltpu.sync_copy(w_hbm, w_v)
    pltpu.sync_copy(b_hbm, b_v)

    vrow = jnp.minimum(lane, 9)
    t0 = plsc.load_gather(tab_v, [vrow * 3])
    t1 = plsc.load_gather(tab_v, [vrow * 3 + 1])
    t2 = plsc.load_gather(tab_v, [vrow * 3 + 2])
    wv = w_v[...]
    lut = (t0 * wv[0] + t1 * wv[1] + t2 * wv[2]) * jnp.float32(1.0 / L)
    for v in range(10):
        lut_v[pl.ds(v * 16, 16)] = jnp.full((16,), lut[v], jnp.float32)

    bias = jnp.full((16,), b_v[...][0], jnp.float32)
    offs = [(lane + d) & 15 for d in range(16)]

    def addword(word, acc_w):
        a0 = (word << 4) & 0xF0
        a1 = (word >> 0) & 0xF0
        a2 = (word >> 4) & 0xF0
        a3 = (word >> 8) & 0xF0
        acc_w = acc_w + plsc.load_gather(lut_v, [a0 + lane])
        acc_w = acc_w + plsc.load_gather(lut_v, [a1 + lane])
        acc_w = acc_w + plsc.load_gather(lut_v, [a2 + lane])
        acc_w = acc_w + plsc.load_gather(lut_v, [a3 + lane])
        return acc_w

    def group(g, carry):
        rowbase = (g * 16 + lane) * WPR
        acc = bias

        def window(w, acc_w):
            rb_w = rowbase + w * 16
            for d in range(16):
                a = rb_w + offs[d]
                word = plsc.load_gather(slab_v, [a >> 7, a & 127])
                acc_w = addword(word, acc_w)
            return acc_w

        acc = lax.fori_loop(0, WPR // 16, window, acc)
        rb_t = rowbase + (WPR // 16) * 16
        for d in range(WPR % 16):
            a = rb_t + (offs[d] & (WPR % 16 - 1))
            word = plsc.load_gather(slab_v, [a >> 7, a & 127])
            acc = addword(word, acc)
        out_v[pl.ds(g * 16, 16)] = acc
        return carry

    for cp in cps:
        cp.wait()
    lax.fori_loop(0, GROUPS, group, 0)

    pltpu.sync_copy(out_v, out_hbm.at[pl.ds(wid * ROWS_PER_TILE,
                                            ROWS_PER_TILE)])


@jax.jit
def _run(packed2d, table_flat, w_pad, b_pad):
    mesh = plsc.VectorSubcoreMesh(core_axis_name="c", subcore_axis_name="s")
    fn = pl.kernel(
        _body,
        out_type=jax.ShapeDtypeStruct((B,), jnp.float32),
        mesh=mesh,
        scratch_types=[
            pltpu.VMEM((DMA_ROWS, DROW), jnp.int32),
            pltpu.VMEM((ROWS_PER_TILE,), jnp.float32),
            pltpu.VMEM((30,), jnp.float32),
            pltpu.VMEM((16,), jnp.float32),
            pltpu.VMEM((16,), jnp.float32),
            pltpu.VMEM((160,), jnp.float32),
            pltpu.VMEM((ILIST_PAD,), jnp.int32),
            pltpu.SemaphoreType.DMA,
        ],
        compiler_params=pltpu.CompilerParams(needs_layout_passes=False),
    )
    return fn(packed2d, table_flat, w_pad, b_pad)


def kernel(inputs, table, W, b):
    packed = jnp.dot(inputs.astype(jnp.float32), jnp.asarray(_PACK),
                     preferred_element_type=jnp.float32)
    packed2d = packed.astype(jnp.int32).reshape(B * WPR // DROW, DROW)
    w_pad = jnp.zeros((16,), jnp.float32).at[:3].set(W.reshape(-1))
    b_pad = jnp.zeros((16,), jnp.float32).at[:1].set(b)
    out = _run(packed2d, table.reshape(-1), w_pad, b_pad)
    return out.reshape(B, 1)
